# baseline (speedup 1.0000x reference)
import functools

import jax
import jax.numpy as jnp
from jax import lax
from jax.experimental import pallas as pl
from jax.experimental.pallas import tpu as pltpu
from jax.experimental.pallas import tpu_sc as plsc

_NC = 2
_NS = 16


def _seg_sum_sc(f0, f1, src_r, dst_r, zeros_acc, *, n, e, b):
    chunks = e // b
    rows_per_tile = chunks // _NS
    n_per_tile = n // _NS
    mesh = plsc.VectorSubcoreMesh(core_axis_name="c", subcore_axis_name="s",
                                  num_cores=_NC, num_subcores=_NS)

    out_type = [
        jax.ShapeDtypeStruct((n, 128), jnp.float32),
        jax.ShapeDtypeStruct((n, 128), jnp.float32),
    ]

    rows_half = rows_per_tile // 2
    scratch = [
        pltpu.VMEM((rows_half, b), jnp.int32),
        pltpu.VMEM((rows_half, b), jnp.int32),
        pltpu.VMEM((b, 128), jnp.float32),
        pltpu.VMEM((b, 128), jnp.float32),
        pltpu.VMEM_SHARED((n, 128), jnp.float32),
        pltpu.SemaphoreType.DMA,
        pltpu.SemaphoreType.DMA,
    ]

    def body(f0_hbm, f1_hbm, src_hbm, dst_hbm, z_hbm, out0, out1,
             src_v, dst_v, rows_a, rows_b, acc, sem_a, sem_b):
        c = lax.axis_index("c")
        s = lax.axis_index("s")
        r0 = s * n_per_tile

        pltpu.sync_copy(z_hbm.at[pl.ds(r0, n_per_tile)],
                        acc.at[pl.ds(r0, n_per_tile)])
        plsc.subcore_barrier()

        def run(f_hbm):
            for p in range(2):
                row0 = s * rows_per_tile + p * rows_half
                pltpu.sync_copy(src_hbm.at[pl.ds(row0, rows_half)], src_v)
                pltpu.sync_copy(dst_hbm.at[pl.ds(row0, rows_half)], dst_v)

                pltpu.async_copy(f_hbm.at[src_v.at[0]], rows_a, sem_a)

                def step(k, carry):
                    i = 2 * k
                    pltpu.async_copy(f_hbm.at[src_v.at[i + 1]], rows_b, sem_b)
                    pltpu.make_async_copy(f_hbm.at[src_v.at[i]], rows_a,
                                          sem_a).wait()
                    pltpu.sync_copy(rows_a, acc.at[dst_v.at[i]], add=True)

                    @pl.when(i + 2 < rows_half)
                    def _():
                        pltpu.async_copy(f_hbm.at[src_v.at[i + 2]], rows_a,
                                         sem_a)

                    pltpu.make_async_copy(f_hbm.at[src_v.at[i + 1]], rows_b,
                                          sem_b).wait()
                    pltpu.sync_copy(rows_b, acc.at[dst_v.at[i + 1]], add=True)
                    return carry

                lax.fori_loop(0, rows_half // 2, step, 0)

        @pl.when(c == 0)
        def _():
            run(f0_hbm)

        @pl.when(c == 1)
        def _():
            run(f1_hbm)

        plsc.subcore_barrier()

        @pl.when(c == 0)
        def _():
            pltpu.sync_copy(acc.at[pl.ds(r0, n_per_tile)],
                            out0.at[pl.ds(r0, n_per_tile)])

        @pl.when(c == 1)
        def _():
            pltpu.sync_copy(acc.at[pl.ds(r0, n_per_tile)],
                            out1.at[pl.ds(r0, n_per_tile)])

    fn = pl.kernel(body, out_type=out_type, mesh=mesh, scratch_types=scratch)
    return fn(f0, f1, src_r, dst_r, zeros_acc)


def _dense_tc(s0, s1, f0, f1, W_l, b_l, W_r, *, n, relu, split_out):
    bs = 400
    grid = (n // bs,)
    b2 = b_l.reshape(1, -1)
    d = W_l.shape[1]
    h = d // 2

    W_cat = jnp.concatenate([W_l, W_r], axis=1)

    def body(s0_ref, s1_ref, f0_ref, f1_ref, wc_ref, bl_ref, o_ref, *rest):
        mean = jnp.concatenate([s0_ref[...], s1_ref[...]], axis=1)
        nrm = jnp.sqrt(jnp.sum(mean * mean, axis=1, keepdims=True))
        mean = mean / jnp.maximum(nrm, 1e-12)
        xr = jnp.concatenate([f0_ref[...], f1_ref[...]], axis=1)
        xn = jnp.sqrt(jnp.sum(xr * xr, axis=1, keepdims=True))
        xr = xr / jnp.maximum(xn, 1e-12)
        xcat = jnp.concatenate([mean, xr], axis=1).astype(jnp.bfloat16)
        out = lax.dot_general(xcat, wc_ref[...].astype(jnp.bfloat16),
                              (((1,), (1,)), ((), ())),
                              preferred_element_type=jnp.float32)
        out = out + bl_ref[...]
        if relu:
            out = jnp.maximum(out, 0.0)
        if split_out:
            o_ref[...] = out[:, :h]
            rest[0][...] = out[:, h:]
        else:
            o_ref[...] = out

    half_spec = pl.BlockSpec((bs, h), lambda i: (i, 0))
    in_specs = [
        half_spec, half_spec, half_spec, half_spec,
        pl.BlockSpec((d, 2 * d), lambda i: (0, 0)),
        pl.BlockSpec((1, d), lambda i: (0, 0)),
    ]
    if split_out:
        out_shape = [jax.ShapeDtypeStruct((n, h), jnp.float32),
                     jax.ShapeDtypeStruct((n, h), jnp.float32)]
        out_specs = [half_spec, half_spec]
    else:
        out_shape = jax.ShapeDtypeStruct((n, d), jnp.float32)
        out_specs = pl.BlockSpec((bs, d), lambda i: (i, 0))
    return pl.pallas_call(
        body, grid=grid, in_specs=in_specs, out_specs=out_specs,
        out_shape=out_shape,
    )(s0, s1, f0, f1, W_cat, b2)


def kernel(x, edge_index, W_l1, b_l1, W_r1, W_l2, b_l2, W_r2):
    n, d = x.shape
    e = edge_index.shape[1]
    h = d // 2
    b = 125
    n_pad = ((n + 8 * _NS - 1) // (8 * _NS)) * (8 * _NS)

    src = edge_index[0].astype(jnp.int32)
    dst = edge_index[1].astype(jnp.int32)
    src_r = src.reshape(e // b, b)
    dst_r = dst.reshape(e // b, b)

    x0 = x[:, :h]
    x1 = x[:, h:]
    zeros_acc = jnp.zeros((n_pad, 128), jnp.float32)

    s0, s1 = _seg_sum_sc(x0, x1, src_r, dst_r, zeros_acc, n=n_pad, e=e, b=b)
    h0, h1 = _dense_tc(s0, s1, x0, x1, W_l1, b_l1, W_r1,
                       n=n, relu=True, split_out=True)
    t0, t1 = _seg_sum_sc(h0, h1, src_r, dst_r, zeros_acc, n=n_pad, e=e, b=b)
    out = _dense_tc(t0, t1, h0, h1, W_l2, b_l2, W_r2,
                    n=n, relu=False, split_out=False)
    return out


# speedup vs baseline: 1.0697x; 1.0697x over previous
import functools

import jax
import jax.numpy as jnp
from jax import lax
from jax.experimental import pallas as pl
from jax.experimental.pallas import tpu as pltpu
from jax.experimental.pallas import tpu_sc as plsc

_NC = 2
_NS = 16


def _seg_sum_sc(f, edge_r, zeros_acc, *, n, e, b):
    chunks = e // b
    rows_per_tile = chunks // _NS
    n_per_tile = n // _NS
    mesh = plsc.VectorSubcoreMesh(core_axis_name="c", subcore_axis_name="s",
                                  num_cores=_NC, num_subcores=_NS)

    out_type = jax.ShapeDtypeStruct((n, 256), jnp.float32)

    rows_half = rows_per_tile // 2
    scratch = [
        pltpu.VMEM((rows_half, b), jnp.int32),
        pltpu.VMEM((rows_half, b), jnp.int32),
        pltpu.VMEM((b, 128), jnp.float32),
        pltpu.VMEM((b, 128), jnp.float32),
        pltpu.VMEM_SHARED((n, 128), jnp.float32),
        pltpu.SemaphoreType.DMA,
        pltpu.SemaphoreType.DMA,
    ]

    def body(f_hbm, edge_hbm, z_hbm, out,
             src_v, dst_v, rows_a, rows_b, acc, sem_a, sem_b):
        c = lax.axis_index("c")
        s = lax.axis_index("s")
        r0 = s * n_per_tile

        pltpu.sync_copy(z_hbm.at[pl.ds(r0, n_per_tile)],
                        acc.at[pl.ds(r0, n_per_tile)])
        plsc.subcore_barrier()

        def run(col0):
            fcol = f_hbm.at[:, pl.ds(col0, 128)]
            for p in range(2):
                row0 = s * rows_per_tile + p * rows_half
                pltpu.sync_copy(edge_hbm.at[pl.ds(row0, rows_half)], src_v)
                pltpu.sync_copy(edge_hbm.at[pl.ds(chunks + row0, rows_half)],
                                dst_v)

                pltpu.async_copy(fcol.at[src_v.at[0]], rows_a, sem_a)

                def step(k, carry):
                    i = 2 * k
                    pltpu.async_copy(fcol.at[src_v.at[i + 1]], rows_b, sem_b)
                    pltpu.make_async_copy(fcol.at[src_v.at[i]], rows_a,
                                          sem_a).wait()
                    pltpu.sync_copy(rows_a, acc.at[dst_v.at[i]], add=True)

                    @pl.when(i + 2 < rows_half)
                    def _():
                        pltpu.async_copy(fcol.at[src_v.at[i + 2]], rows_a,
                                         sem_a)

                    pltpu.make_async_copy(fcol.at[src_v.at[i + 1]], rows_b,
                                          sem_b).wait()
                    pltpu.sync_copy(rows_b, acc.at[dst_v.at[i + 1]], add=True)
                    return carry

                lax.fori_loop(0, rows_half // 2, step, 0)

        @pl.when(c == 0)
        def _():
            run(0)

        @pl.when(c == 1)
        def _():
            run(128)

        plsc.subcore_barrier()

        @pl.when(c == 0)
        def _():
            pltpu.sync_copy(acc.at[pl.ds(r0, n_per_tile)],
                            out.at[pl.ds(r0, n_per_tile), pl.ds(0, 128)])

        @pl.when(c == 1)
        def _():
            pltpu.sync_copy(acc.at[pl.ds(r0, n_per_tile)],
                            out.at[pl.ds(r0, n_per_tile), pl.ds(128, 128)])

    fn = pl.kernel(body, out_type=out_type, mesh=mesh, scratch_types=scratch)
    return fn(f, edge_r, zeros_acc)


def _dense_tc(sseg, f, W_l, b_l, W_r, *, n, relu):
    bs = 1000
    grid = (n // bs,)
    b2 = b_l.reshape(1, -1)
    d = W_l.shape[1]

    W_cat = jnp.concatenate([W_l, W_r], axis=1)

    def body(s_ref, f_ref, wc_ref, bl_ref, o_ref):
        mean = s_ref[...]
        nrm = jnp.sqrt(jnp.sum(mean * mean, axis=1, keepdims=True))
        mean = mean / jnp.maximum(nrm, 1e-12)
        xr = f_ref[...]
        xn = jnp.sqrt(jnp.sum(xr * xr, axis=1, keepdims=True))
        xr = xr / jnp.maximum(xn, 1e-12)
        xcat = jnp.concatenate([mean, xr], axis=1)
        out = lax.dot_general(xcat, wc_ref[...], (((1,), (1,)), ((), ())),
                              preferred_element_type=jnp.float32)
        out = out + bl_ref[...]
        if relu:
            out = jnp.maximum(out, 0.0)
        o_ref[...] = out

    blk = pl.BlockSpec((bs, d), lambda i: (i, 0))
    in_specs = [
        blk, blk,
        pl.BlockSpec((d, 2 * d), lambda i: (0, 0)),
        pl.BlockSpec((1, d), lambda i: (0, 0)),
    ]
    return pl.pallas_call(
        body, grid=grid, in_specs=in_specs, out_specs=blk,
        out_shape=jax.ShapeDtypeStruct((n, d), jnp.float32),
    )(sseg, f, W_cat, b2)


def kernel(x, edge_index, W_l1, b_l1, W_r1, W_l2, b_l2, W_r2):
    n, d = x.shape
    e = edge_index.shape[1]
    b = 125
    chunks = e // b
    n_pad = ((n + 8 * _NS - 1) // (8 * _NS)) * (8 * _NS)

    edge_r = edge_index.astype(jnp.int32).reshape(2 * chunks, b)
    zeros_acc = jnp.zeros((n_pad, 128), jnp.float32)

    s1 = _seg_sum_sc(x, edge_r, zeros_acc, n=n_pad, e=e, b=b)
    h1 = _dense_tc(s1, x, W_l1, b_l1, W_r1, n=n, relu=True)
    s2 = _seg_sum_sc(h1, edge_r, zeros_acc, n=n_pad, e=e, b=b)
    out = _dense_tc(s2, h1, W_l2, b_l2, W_r2, n=n, relu=False)
    return out


# speedup vs baseline: 1.0921x; 1.0209x over previous
import functools

import jax
import jax.numpy as jnp
from jax import lax
from jax.experimental import pallas as pl
from jax.experimental.pallas import tpu as pltpu
from jax.experimental.pallas import tpu_sc as plsc

_NC = 2
_NS = 16


def _seg_sum_sc(f, edge_r, zeros_acc, *, n, e, b):
    chunks = e // b
    rows_per_tile = chunks // _NS
    n_per_tile = n // _NS
    mesh = plsc.VectorSubcoreMesh(core_axis_name="c", subcore_axis_name="s",
                                  num_cores=_NC, num_subcores=_NS)

    out_type = jax.ShapeDtypeStruct((n, 256), jnp.float32)

    phase_len = 40
    n_phases = rows_per_tile // phase_len
    scratch = [
        pltpu.VMEM((phase_len, b), jnp.int32),
        pltpu.VMEM((phase_len, b), jnp.int32),
        pltpu.VMEM((b, 128), jnp.float32),
        pltpu.VMEM((b, 128), jnp.float32),
        pltpu.VMEM((b, 128), jnp.float32),
        pltpu.VMEM((b, 128), jnp.float32),
        pltpu.VMEM_SHARED((n, 128), jnp.float32),
        pltpu.SemaphoreType.DMA,
        pltpu.SemaphoreType.DMA,
        pltpu.SemaphoreType.DMA,
        pltpu.SemaphoreType.DMA,
        pltpu.SemaphoreType.DMA,
        pltpu.SemaphoreType.DMA,
        pltpu.SemaphoreType.DMA,
        pltpu.SemaphoreType.DMA,
    ]

    def body(f_hbm, edge_hbm, z_hbm, out, src_v, dst_v,
             rows_0, rows_1, rows_2, rows_3, acc,
             gs_0, gs_1, gs_2, gs_3, ss_0, ss_1, ss_2, ss_3):
        c = lax.axis_index("c")
        s = lax.axis_index("s")
        r0 = s * n_per_tile
        bufs = (rows_0, rows_1, rows_2, rows_3)
        gs = (gs_0, gs_1, gs_2, gs_3)
        ss = (ss_0, ss_1, ss_2, ss_3)

        pltpu.sync_copy(z_hbm.at[pl.ds(r0, n_per_tile)],
                        acc.at[pl.ds(r0, n_per_tile)])
        plsc.subcore_barrier()

        def run(col0):
            fcol = f_hbm.at[:, pl.ds(col0, 128)]

            def gather(i, q):
                pltpu.async_copy(fcol.at[src_v.at[i]], bufs[q], gs[q])

            def scatter_start(i, q):
                pltpu.async_copy(bufs[q], acc.at[dst_v.at[i]], ss[q],
                                 add=True)

            def scatter_wait(i, q):
                pltpu.make_async_copy(bufs[q], acc.at[dst_v.at[i]],
                                      ss[q]).wait()

            def phase(p, carry):
                row0 = pl.multiple_of(s * rows_per_tile + p * phase_len, 8)
                pltpu.sync_copy(edge_hbm.at[pl.ds(row0, phase_len)], src_v)
                pltpu.sync_copy(edge_hbm.at[pl.ds(chunks + row0, phase_len)],
                                dst_v)

                gather(0, 0)
                gather(1, 1)

                def step(k, carry):
                    for j in range(4):
                        i = 4 * k + j
                        qn = (j + 2) % 4

                        @pl.when(i + 2 < phase_len)
                        def _():
                            @pl.when(i >= 2)
                            def _():
                                scatter_wait(i - 2, qn)

                            gather(i + 2, qn)

                        pltpu.make_async_copy(fcol.at[src_v.at[i]], bufs[j],
                                              gs[j]).wait()
                        scatter_start(i, j)
                    return carry

                lax.fori_loop(0, phase_len // 4, step, 0)
                for q in range(4):
                    scatter_wait(phase_len - 4 + q, q)
                return carry

            lax.fori_loop(0, n_phases, phase, 0)

        @pl.when(c == 0)
        def _():
            run(0)

        @pl.when(c == 1)
        def _():
            run(128)

        plsc.subcore_barrier()

        @pl.when(c == 0)
        def _():
            pltpu.sync_copy(acc.at[pl.ds(r0, n_per_tile)],
                            out.at[pl.ds(r0, n_per_tile), pl.ds(0, 128)])

        @pl.when(c == 1)
        def _():
            pltpu.sync_copy(acc.at[pl.ds(r0, n_per_tile)],
                            out.at[pl.ds(r0, n_per_tile), pl.ds(128, 128)])

    fn = pl.kernel(body, out_type=out_type, mesh=mesh, scratch_types=scratch)
    return fn(f, edge_r, zeros_acc)


def _dense_tc(sseg, f, W_l, b_l, W_r, *, n, relu):
    bs = 1000
    grid = (n // bs,)
    b2 = b_l.reshape(1, -1)
    d = W_l.shape[1]

    W_cat = jnp.concatenate([W_l, W_r], axis=1)

    def body(s_ref, f_ref, wc_ref, bl_ref, o_ref):
        mean = s_ref[...]
        nrm = jnp.sqrt(jnp.sum(mean * mean, axis=1, keepdims=True))
        mean = mean / jnp.maximum(nrm, 1e-12)
        xr = f_ref[...]
        xn = jnp.sqrt(jnp.sum(xr * xr, axis=1, keepdims=True))
        xr = xr / jnp.maximum(xn, 1e-12)
        xcat = jnp.concatenate([mean, xr], axis=1)
        out = lax.dot_general(xcat, wc_ref[...], (((1,), (1,)), ((), ())),
                              preferred_element_type=jnp.float32)
        out = out + bl_ref[...]
        if relu:
            out = jnp.maximum(out, 0.0)
        o_ref[...] = out

    blk = pl.BlockSpec((bs, d), lambda i: (i, 0))
    in_specs = [
        blk, blk,
        pl.BlockSpec((d, 2 * d), lambda i: (0, 0)),
        pl.BlockSpec((1, d), lambda i: (0, 0)),
    ]
    return pl.pallas_call(
        body, grid=grid, in_specs=in_specs, out_specs=blk,
        out_shape=jax.ShapeDtypeStruct((n, d), jnp.float32),
    )(sseg, f, W_cat, b2)


def kernel(x, edge_index, W_l1, b_l1, W_r1, W_l2, b_l2, W_r2):
    n, d = x.shape
    e = edge_index.shape[1]
    b = 50
    chunks = e // b
    n_pad = ((n + 8 * _NS - 1) // (8 * _NS)) * (8 * _NS)

    edge_r = edge_index.astype(jnp.int32).reshape(2 * chunks, b)
    zeros_acc = jnp.zeros((n_pad, 128), jnp.float32)

    s1 = _seg_sum_sc(x, edge_r, zeros_acc, n=n_pad, e=e, b=b)
    h1 = _dense_tc(s1, x, W_l1, b_l1, W_r1, n=n, relu=True)
    s2 = _seg_sum_sc(h1, edge_r, zeros_acc, n=n_pad, e=e, b=b)
    out = _dense_tc(s2, h1, W_l2, b_l2, W_r2, n=n, relu=False)
    return out
